# baseline (speedup 1.0000x reference)
import functools

import jax
import jax.numpy as jnp
from jax import lax
from jax.experimental import pallas as pl
from jax.experimental.pallas import tpu as pltpu
from jax.experimental.pallas import tpu_sc as plsc

_S = 200
_BB = 128


@functools.lru_cache(maxsize=None)
def _make_gather(batch: int, seq: int, vocab: int, d: int):
    info = plsc.get_sparse_core_info()
    nw = info.num_cores * info.num_subcores
    assert seq == _S and d == 64 and batch == _BB * nw

    mesh = plsc.VectorSubcoreMesh(core_axis_name="c", subcore_axis_name="s")

    @functools.partial(
        pl.kernel,
        mesh=mesh,
        out_type=jax.ShapeDtypeStruct((seq, 8, batch // 128, 8, 128),
                                      jnp.float32),
        scratch_types=[
            *[pltpu.VMEM((_BB,), jnp.int32) for _ in range(2)],
            *[pltpu.VMEM((_BB, d), jnp.float32) for _ in range(2)],
            *[pltpu.VMEM((8, 8, 128), jnp.float32) for _ in range(2)],
            *[pltpu.SemaphoreType.DMA for _ in range(6)],
        ],
        compiler_params=pltpu.CompilerParams(
            use_tc_tiling_on_sc=False, needs_layout_passes=False),
    )
    def k(idx_hbm, table_hbm, out_hbm, i0, i1, r0, r1, t0, t1,
          is0, is1, gs0, gs1, os0, os1):
        idxb = (i0, i1)
        raw = (r0, r1)
        tr = (t0, t1)
        isem = (is0, is1)
        gsem = (gs0, gs1)
        osem = (os0, os1)
        wid = lax.axis_index("s") * info.num_cores + lax.axis_index("c")
        col0 = wid * _BB

        lanes = lax.iota(jnp.int32, 16)
        rows_g = [g * 16 + lanes for g in range(8)]

        def i_desc(t, slot):
            return pltpu.make_async_copy(
                idx_hbm.at[t, pl.ds(col0, _BB)], idxb[slot], isem[slot])

        def g_desc(slot):
            return pltpu.make_async_copy(
                table_hbm.at[idxb[slot]], raw[slot], gsem[slot])

        def o_desc(t, slot):
            return pltpu.make_async_copy(
                tr[slot], out_hbm.at[t, :, wid], osem[slot])

        def transpose(slot):
            rawb = raw[slot]
            trb = tr[slot]

            def dq_body(dq, carry):
                for dr in range(8):
                    col = jnp.broadcast_to(dq * 8 + dr, (16,)).astype(
                        jnp.int32)
                    for g in range(8):
                        v = plsc.load_gather(rawb, [rows_g[g], col])
                        trb[dq, dr, pl.ds(g * 16, 16)] = v
                return carry

            lax.fori_loop(0, 8, dq_body, 0)

        i_desc(0, 0).start()
        i_desc(1, 1).start()
        i_desc(0, 0).wait()
        g_desc(0).start()

        def full_step(t, slot, *, sg_next, si_next, wo_prev):
            g_desc(slot).wait()
            if sg_next is not None:
                i_desc(sg_next, 1 - slot).wait()
                g_desc(1 - slot).start()
            if si_next is not None:
                i_desc(si_next, slot).start()
            if wo_prev is not None:
                o_desc(wo_prev, 1 - slot).wait()
            transpose(slot)
            o_desc(t, slot).start()

        full_step(0, 0, sg_next=1, si_next=2, wo_prev=None)
        full_step(1, 1, sg_next=2, si_next=3, wo_prev=0)

        def body(tp, carry):
            t = 2 + tp * 2
            full_step(t, 0, sg_next=t + 1, si_next=t + 2, wo_prev=t - 1)
            full_step(t + 1, 1, sg_next=t + 2, si_next=t + 3,
                      wo_prev=t)
            return carry

        lax.fori_loop(0, (seq - 4) // 2, body, 0)

        full_step(seq - 2, 0, sg_next=seq - 1, si_next=None,
                  wo_prev=seq - 3)
        full_step(seq - 1, 1, sg_next=None, si_next=None,
                  wo_prev=seq - 2)
        o_desc(seq - 1, 1).wait()

    return k


def kernel(words_seq, table):
    b, s = words_seq.shape
    v, d = table.shape
    idx_t = words_seq.T.astype(jnp.int32)
    out5 = _make_gather(b, s, v, d)(idx_t, table)
    return out5.transpose(2, 4, 0, 1, 3).reshape(b, s, d)


# speedup vs baseline: 1.6986x; 1.6986x over previous
import functools

import jax
import jax.numpy as jnp
from jax import lax
from jax.experimental import pallas as pl
from jax.experimental.pallas import tpu as pltpu
from jax.experimental.pallas import tpu_sc as plsc

_S = 200
_BB = 128


@functools.lru_cache(maxsize=None)
def _make_gather(batch: int, seq: int, vocab: int, d: int):
    info = plsc.get_sparse_core_info()
    nw = info.num_cores * info.num_subcores
    assert seq == _S and d == 64 and batch == _BB * nw

    mesh = plsc.VectorSubcoreMesh(core_axis_name="c", subcore_axis_name="s")

    @functools.partial(
        pl.kernel,
        mesh=mesh,
        out_type=jax.ShapeDtypeStruct((seq, 8, batch // 128, 8, 128),
                                      jnp.float32),
        scratch_types=[
            *[pltpu.VMEM((_BB,), jnp.int32) for _ in range(2)],
            *[pltpu.VMEM((_BB, d), jnp.float32) for _ in range(2)],
            *[pltpu.VMEM((8, 8, 128), jnp.float32) for _ in range(2)],
            *[pltpu.SemaphoreType.DMA for _ in range(6)],
        ],
        compiler_params=pltpu.CompilerParams(
            use_tc_tiling_on_sc=False, needs_layout_passes=False),
    )
    def k(idx_hbm, table_hbm, out_hbm, i0, i1, r0, r1, t0, t1,
          is0, is1, gs0, gs1, os0, os1):
        idxb = (i0, i1)
        raw = (r0, r1)
        tr = (t0, t1)
        isem = (is0, is1)
        gsem = (gs0, gs1)
        osem = (os0, os1)
        wid = lax.axis_index("s") * info.num_cores + lax.axis_index("c")
        col0 = wid * _BB

        lanes = lax.iota(jnp.int32, 16)
        rows_g = [g * 16 + lanes for g in range(8)]

        def i_desc(t, slot):
            return pltpu.make_async_copy(
                idx_hbm.at[t, pl.ds(col0, _BB)], idxb[slot], isem[slot])

        def g_desc(slot):
            return pltpu.make_async_copy(
                table_hbm.at[idxb[slot]], raw[slot], gsem[slot])

        def o_desc(t, slot):
            return pltpu.make_async_copy(
                tr[slot], out_hbm.at[t, :, wid], osem[slot])

        def transpose(slot):
            rawb = raw[slot]
            trb = tr[slot]

            @plsc.parallel_loop(0, 64, unroll=8)
            def col_body(d):
                dq = d // 8
                dr = d % 8
                col = jnp.broadcast_to(d, (16,)).astype(jnp.int32)
                for g in range(8):
                    v = plsc.load_gather(rawb, [rows_g[g], col])
                    trb[dq, dr, pl.ds(g * 16, 16)] = v

        i_desc(0, 0).start()
        i_desc(1, 1).start()
        i_desc(0, 0).wait()
        g_desc(0).start()

        def full_step(t, slot, *, sg_next, si_next, wo_prev):
            g_desc(slot).wait()
            if sg_next is not None:
                i_desc(sg_next, 1 - slot).wait()
                g_desc(1 - slot).start()
            if si_next is not None:
                i_desc(si_next, slot).start()
            if wo_prev is not None:
                o_desc(wo_prev, 1 - slot).wait()
            transpose(slot)
            o_desc(t, slot).start()

        full_step(0, 0, sg_next=1, si_next=2, wo_prev=None)
        full_step(1, 1, sg_next=2, si_next=3, wo_prev=0)

        def body(tp, carry):
            t = 2 + tp * 2
            full_step(t, 0, sg_next=t + 1, si_next=t + 2, wo_prev=t - 1)
            full_step(t + 1, 1, sg_next=t + 2, si_next=t + 3,
                      wo_prev=t)
            return carry

        lax.fori_loop(0, (seq - 4) // 2, body, 0)

        full_step(seq - 2, 0, sg_next=seq - 1, si_next=None,
                  wo_prev=seq - 3)
        full_step(seq - 1, 1, sg_next=None, si_next=None,
                  wo_prev=seq - 2)
        o_desc(seq - 1, 1).wait()

    return k


def kernel(words_seq, table):
    b, s = words_seq.shape
    v, d = table.shape
    idx_t = words_seq.T.astype(jnp.int32)
    out5 = _make_gather(b, s, v, d)(idx_t, table)
    return out5.transpose(2, 4, 0, 1, 3).reshape(b, s, d)


# speedup vs baseline: 4.4784x; 2.6366x over previous
import functools

import jax
import jax.numpy as jnp
from jax import lax
from jax.experimental import pallas as pl
from jax.experimental.pallas import tpu as pltpu
from jax.experimental.pallas import tpu_sc as plsc

_S = 200
_BB = 128


@functools.lru_cache(maxsize=None)
def _make_gather(batch: int, seq: int, vocab: int, d: int):
    info = plsc.get_sparse_core_info()
    nw = info.num_cores * info.num_subcores
    assert seq == _S and d == 64 and batch == _BB * nw

    mesh = plsc.VectorSubcoreMesh(core_axis_name="c", subcore_axis_name="s")

    @functools.partial(
        pl.kernel,
        mesh=mesh,
        out_type=jax.ShapeDtypeStruct((seq, 8, batch // 128, 8, 128),
                                      jnp.float32),
        scratch_types=[
            *[pltpu.VMEM((_BB,), jnp.int32) for _ in range(2)],
            *[pltpu.VMEM((_BB, d), jnp.float32) for _ in range(2)],
            *[pltpu.VMEM((8, 8, 129), jnp.float32) for _ in range(2)],
            *[pltpu.SemaphoreType.DMA for _ in range(6)],
        ],
        compiler_params=pltpu.CompilerParams(
            use_tc_tiling_on_sc=False, needs_layout_passes=False),
    )
    def k(idx_hbm, table_hbm, out_hbm, i0, i1, r0, r1, t0, t1,
          is0, is1, gs0, gs1, os0, os1):
        idxb = (i0, i1)
        raw = (r0, r1)
        tr = (t0, t1)
        isem = (is0, is1)
        gsem = (gs0, gs1)
        osem = (os0, os1)
        wid = lax.axis_index("s") * info.num_cores + lax.axis_index("c")
        col0 = wid * _BB

        lanes = lax.iota(jnp.int32, 16)
        dq_vecs = [(k * 16 + lanes) // 8 for k in range(4)]
        dr_vecs = [(k * 16 + lanes) % 8 for k in range(4)]

        def i_desc(t, slot):
            return pltpu.make_async_copy(
                idx_hbm.at[t, pl.ds(col0, _BB)], idxb[slot], isem[slot])

        def g_desc(slot):
            return pltpu.make_async_copy(
                table_hbm.at[idxb[slot]], raw[slot], gsem[slot])

        def o_desc(t, slot):
            return pltpu.make_async_copy(
                tr[slot].at[:, :, pl.ds(0, 128)], out_hbm.at[t, :, wid],
                osem[slot])

        def transpose(slot):
            rawb = raw[slot]
            trb = tr[slot]

            @plsc.parallel_loop(0, _BB, unroll=8)
            def b_body(b):
                bcol = jnp.broadcast_to(b, (16,)).astype(jnp.int32)
                for k in range(4):
                    v = rawb[b, pl.ds(k * 16, 16)]
                    plsc.store_scatter(
                        trb, [dq_vecs[k], dr_vecs[k], bcol], v)

        i_desc(0, 0).start()
        i_desc(1, 1).start()
        i_desc(0, 0).wait()
        g_desc(0).start()

        def full_step(t, slot, *, sg_next, si_next, wo_prev):
            g_desc(slot).wait()
            if sg_next is not None:
                i_desc(sg_next, 1 - slot).wait()
                g_desc(1 - slot).start()
            if si_next is not None:
                i_desc(si_next, slot).start()
            if wo_prev is not None:
                o_desc(wo_prev, 1 - slot).wait()
            transpose(slot)
            o_desc(t, slot).start()

        full_step(0, 0, sg_next=1, si_next=2, wo_prev=None)
        full_step(1, 1, sg_next=2, si_next=3, wo_prev=0)

        def body(tp, carry):
            t = 2 + tp * 2
            full_step(t, 0, sg_next=t + 1, si_next=t + 2, wo_prev=t - 1)
            full_step(t + 1, 1, sg_next=t + 2, si_next=t + 3,
                      wo_prev=t)
            return carry

        lax.fori_loop(0, (seq - 4) // 2, body, 0)

        full_step(seq - 2, 0, sg_next=seq - 1, si_next=None,
                  wo_prev=seq - 3)
        full_step(seq - 1, 1, sg_next=None, si_next=None,
                  wo_prev=seq - 2)
        o_desc(seq - 1, 1).wait()

    return k


def kernel(words_seq, table):
    b, s = words_seq.shape
    v, d = table.shape
    idx_t = words_seq.T.astype(jnp.int32)
    out5 = _make_gather(b, s, v, d)(idx_t, table)
    return out5.transpose(2, 4, 0, 1, 3).reshape(b, s, d)


# speedup vs baseline: 4.5787x; 1.0224x over previous
import functools

import jax
import jax.numpy as jnp
from jax import lax
from jax.experimental import pallas as pl
from jax.experimental.pallas import tpu as pltpu
from jax.experimental.pallas import tpu_sc as plsc

_S = 200
_BB = 128


@functools.lru_cache(maxsize=None)
def _make_gather(batch: int, seq: int, vocab: int, d: int):
    info = plsc.get_sparse_core_info()
    nw = info.num_cores * info.num_subcores
    assert seq == _S and d == 64 and batch == _BB * nw

    mesh = plsc.VectorSubcoreMesh(core_axis_name="c", subcore_axis_name="s")

    @functools.partial(
        pl.kernel,
        mesh=mesh,
        out_type=jax.ShapeDtypeStruct((seq, 8, batch // 128, 8, 128),
                                      jnp.float32),
        scratch_types=[
            *[pltpu.VMEM((_BB,), jnp.int32) for _ in range(2)],
            *[pltpu.VMEM((_BB, d), jnp.float32) for _ in range(2)],
            *[pltpu.VMEM((8, 8, 129), jnp.float32) for _ in range(2)],
            *[pltpu.SemaphoreType.DMA for _ in range(6)],
        ],
        compiler_params=pltpu.CompilerParams(
            use_tc_tiling_on_sc=False, needs_layout_passes=False),
    )
    def k(idx_hbm, table_hbm, out_hbm, i0, i1, r0, r1, t0, t1,
          is0, is1, gs0, gs1, os0, os1):
        idxb = (i0, i1)
        raw = (r0, r1)
        tr = (t0, t1)
        isem = (is0, is1)
        gsem = (gs0, gs1)
        osem = (os0, os1)
        wid = lax.axis_index("s") * info.num_cores + lax.axis_index("c")
        col0 = wid * _BB

        lanes = lax.iota(jnp.int32, 16)
        dq_vecs = [(k * 16 + lanes) // 8 for k in range(4)]
        dr_vecs = [(k * 16 + lanes) % 8 for k in range(4)]

        def i_desc(t, slot):
            return pltpu.make_async_copy(
                idx_hbm.at[t, pl.ds(col0, _BB)], idxb[slot], isem[slot])

        def g_desc(slot):
            return pltpu.make_async_copy(
                table_hbm.at[idxb[slot]], raw[slot], gsem[slot])

        def o_desc(t, slot):
            return pltpu.make_async_copy(
                tr[slot].at[:, :, pl.ds(0, 128)], out_hbm.at[t, :, wid],
                osem[slot])

        def transpose(slot):
            rawb = raw[slot]
            trb = tr[slot]

            @plsc.parallel_loop(0, _BB, unroll=16)
            def b_body(b):
                bcol = jnp.broadcast_to(b, (16,)).astype(jnp.int32)
                for k in range(4):
                    v = rawb[b, pl.ds(k * 16, 16)]
                    plsc.store_scatter(
                        trb, [dq_vecs[k], dr_vecs[k], bcol], v)

        i_desc(0, 0).start()
        i_desc(1, 1).start()
        i_desc(0, 0).wait()
        g_desc(0).start()

        def full_step(t, slot, *, sg_next, si_next, wo_prev):
            g_desc(slot).wait()
            if sg_next is not None:
                i_desc(sg_next, 1 - slot).wait()
                g_desc(1 - slot).start()
            if si_next is not None:
                i_desc(si_next, slot).start()
            if wo_prev is not None:
                o_desc(wo_prev, 1 - slot).wait()
            transpose(slot)
            o_desc(t, slot).start()

        full_step(0, 0, sg_next=1, si_next=2, wo_prev=None)
        full_step(1, 1, sg_next=2, si_next=3, wo_prev=0)

        def body(tp, carry):
            t = 2 + tp * 2
            full_step(t, 0, sg_next=t + 1, si_next=t + 2, wo_prev=t - 1)
            full_step(t + 1, 1, sg_next=t + 2, si_next=t + 3,
                      wo_prev=t)
            return carry

        lax.fori_loop(0, (seq - 4) // 2, body, 0)

        full_step(seq - 2, 0, sg_next=seq - 1, si_next=None,
                  wo_prev=seq - 3)
        full_step(seq - 1, 1, sg_next=None, si_next=None,
                  wo_prev=seq - 2)
        o_desc(seq - 1, 1).wait()

    return k


def kernel(words_seq, table):
    b, s = words_seq.shape
    v, d = table.shape
    table2 = jnp.pad(table, ((0, 0), (0, 64))).reshape(2 * v, d)
    idx_t = words_seq.T.astype(jnp.int32) * 2
    out5 = _make_gather(b, s, 2 * v, d)(idx_t, table2)
    return out5.transpose(2, 4, 0, 1, 3).reshape(b, s, d)


# speedup vs baseline: 5.6267x; 1.2289x over previous
import functools

import jax
import jax.numpy as jnp
from jax import lax
from jax.experimental import pallas as pl
from jax.experimental.pallas import tpu as pltpu
from jax.experimental.pallas import tpu_sc as plsc

_S = 200
_BB = 128
_P = 2
_NS = 3


@functools.lru_cache(maxsize=None)
def _make_gather(batch: int, seq: int, vocab2: int, d: int):
    info = plsc.get_sparse_core_info()
    nw = info.num_cores * info.num_subcores
    assert seq == _S and d == 64 and batch == _BB * nw
    nsteps = seq // _P
    assert (nsteps - 7) % _NS == 0

    mesh = plsc.VectorSubcoreMesh(core_axis_name="c", subcore_axis_name="s")

    @functools.partial(
        pl.kernel,
        mesh=mesh,
        out_type=jax.ShapeDtypeStruct((seq, 8, batch // 128, 8, 128),
                                      jnp.float32),
        scratch_types=[
            *[pltpu.VMEM((_P, _BB), jnp.int32) for _ in range(_NS)],
            *[pltpu.VMEM((_P, _BB, d), jnp.float32) for _ in range(_NS)],
            *[pltpu.VMEM((_P, 8, 8, 129), jnp.float32) for _ in range(_NS)],
            *[pltpu.SemaphoreType.DMA for _ in range(3 * _NS)],
        ],
        compiler_params=pltpu.CompilerParams(
            use_tc_tiling_on_sc=False, needs_layout_passes=False),
    )
    def k(idx_hbm, table_hbm, out_hbm, *refs):
        idxb = refs[0:_NS]
        raw = refs[_NS:2 * _NS]
        tr = refs[2 * _NS:3 * _NS]
        isem = refs[3 * _NS:4 * _NS]
        gsem = refs[4 * _NS:5 * _NS]
        osem = refs[5 * _NS:6 * _NS]
        wid = lax.axis_index("s") * info.num_cores + lax.axis_index("c")
        col0 = wid * _BB

        lanes = lax.iota(jnp.int32, 16)
        dq_vecs = [(kk * 16 + lanes) // 8 for kk in range(4)]
        dr_vecs = [(kk * 16 + lanes) % 8 for kk in range(4)]

        def i_desc(t, slot):
            return pltpu.make_async_copy(
                idx_hbm.at[pl.ds(t * _P, _P), pl.ds(col0, _BB)],
                idxb[slot], isem[slot])

        def g_descs(slot):
            return [
                pltpu.make_async_copy(
                    table_hbm.at[idxb[slot].at[h]], raw[slot].at[h],
                    gsem[slot])
                for h in range(_P)
            ]

        def o_desc(t, slot):
            return pltpu.make_async_copy(
                tr[slot].at[:, :, :, pl.ds(0, 128)],
                out_hbm.at[pl.ds(t * _P, _P), :, wid], osem[slot])

        def transpose(slot):
            for h in range(_P):
                rawb = raw[slot].at[h]
                trb = tr[slot].at[h]

                @plsc.parallel_loop(0, _BB, unroll=16)
                def b_body(b):
                    bcol = jnp.broadcast_to(b, (16,)).astype(jnp.int32)
                    for kk in range(4):
                        v = rawb[b, pl.ds(kk * 16, 16)]
                        plsc.store_scatter(
                            trb, [dq_vecs[kk], dr_vecs[kk], bcol], v)

        def sg(slot):
            for dsc in g_descs(slot):
                dsc.start()

        def wg(slot):
            for dsc in g_descs(slot):
                dsc.wait()

        def step(t, slot, *, g_next=None, i_next=None, o_done=None):
            wg(slot)
            if g_next is not None:
                nslot = (slot + 2) % _NS
                i_desc(g_next, nslot).wait()
                sg(nslot)
            if i_next is not None:
                i_desc(i_next, slot).start()
            if o_done is not None:
                o_desc(o_done, slot).wait()
            transpose(slot)
            o_desc(t, slot).start()

        i_desc(0, 0).start()
        i_desc(1, 1).start()
        i_desc(2, 2).start()
        i_desc(0, 0).wait()
        sg(0)
        i_desc(1, 1).wait()
        sg(1)

        step(0, 0, g_next=2, i_next=3)
        step(1, 1, g_next=3, i_next=4)
        step(2, 2, g_next=4, i_next=5)
        step(3, 0, g_next=5, i_next=6, o_done=0)

        def body(j, carry):
            t = 4 + j * _NS
            step(t, 1, g_next=t + 2, i_next=t + 3, o_done=t - 3)
            step(t + 1, 2, g_next=t + 3, i_next=t + 4, o_done=t - 2)
            step(t + 2, 0, g_next=t + 4, i_next=t + 5, o_done=t - 1)
            return carry

        lax.fori_loop(0, (nsteps - 7) // _NS, body, 0)

        n = nsteps
        step(n - 3, 1, g_next=n - 1, o_done=n - 6)
        step(n - 2, 2, o_done=n - 5)
        step(n - 1, 0, o_done=n - 4)
        o_desc(n - 3, 1).wait()
        o_desc(n - 2, 2).wait()
        o_desc(n - 1, 0).wait()

    return k


def kernel(words_seq, table):
    b, s = words_seq.shape
    v, d = table.shape
    table2 = jnp.pad(table, ((0, 0), (0, 64))).reshape(2 * v, d)
    idx_t = words_seq.T.astype(jnp.int32) * 2
    out5 = _make_gather(b, s, 2 * v, d)(idx_t, table2)
    return out5.transpose(2, 4, 0, 1, 3).reshape(b, s, d)
